# baseline (speedup 1.0000x reference)
import functools

import jax
import jax.numpy as jnp
from jax import lax
from jax.experimental import pallas as pl
from jax.experimental.pallas import tpu as pltpu
from jax.experimental.pallas import tpu_sc as plsc

VOCAB = 100000
DIM = 256
BATCH = 1024
SEQ = 50

NC = 2
NS = 16
NW = NC * NS
BPW = BATCH // NW
LANES = 16

HALF = 128
SEQ2 = 2 * SEQ
SEQ2P = 104

_MESH = plsc.VectorSubcoreMesh(core_axis_name="c", subcore_axis_name="s")


@functools.partial(
    pl.kernel,
    mesh=_MESH,
    out_type=jax.ShapeDtypeStruct((BATCH, DIM), jnp.float32),
    scratch_types=[
        pltpu.VMEM((BPW * SEQ2P,), jnp.int32),
        pltpu.VMEM((SEQ2P, HALF), jnp.float32),
        pltpu.VMEM((SEQ2P, HALF), jnp.float32),
        pltpu.VMEM((BPW, DIM), jnp.float32),
        pltpu.SemaphoreType.DMA,
        pltpu.SemaphoreType.DMA,
    ],
)
def _pool(idx_hbm, table_hbm, h_hbm, idx_v, rows_a, rows_b, hacc_v, sem_a, sem_b):
    wid = lax.axis_index("s") * NC + lax.axis_index("c")
    pltpu.sync_copy(idx_hbm.at[pl.ds(wid * (BPW * SEQ2P), BPW * SEQ2P)], idx_v)

    def idx_slice(r):
        return idx_v.at[pl.ds(r * SEQ2P, SEQ2P)]

    def accum(rows, r):
        def body(j, accs):
            lo = tuple(accs[c] + rows[2 * j, pl.ds(c * LANES, LANES)]
                       for c in range(HALF // LANES))
            hi = tuple(accs[8 + c] + rows[2 * j + 1, pl.ds(c * LANES, LANES)]
                       for c in range(HALF // LANES))
            return lo + hi
        zero = jnp.zeros((LANES,), jnp.float32)
        accs = lax.fori_loop(0, SEQ, body, (zero,) * 16)
        for c in range(HALF // LANES):
            hacc_v[r, pl.ds(c * LANES, LANES)] = accs[c] * (1.0 / SEQ)
            hacc_v[r, pl.ds(HALF + c * LANES, LANES)] = accs[8 + c] * (1.0 / SEQ)

    pltpu.async_copy(table_hbm.at[idx_slice(0)], rows_a, sem_a)

    def loop_body(g, carry):
        r0 = 2 * g
        pltpu.async_copy(table_hbm.at[idx_slice(r0 + 1)], rows_b, sem_b)
        pltpu.make_async_copy(table_hbm.at[idx_slice(r0)], rows_a, sem_a).wait()
        accum(rows_a, r0)

        @pl.when(g < BPW // 2 - 1)
        def _():
            pltpu.async_copy(table_hbm.at[idx_slice(r0 + 2)], rows_a, sem_a)

        pltpu.make_async_copy(table_hbm.at[idx_slice(r0 + 1)], rows_b, sem_b).wait()
        accum(rows_b, r0 + 1)
        return carry

    lax.fori_loop(0, BPW // 2, loop_body, 0)
    pltpu.sync_copy(hacc_v, h_hbm.at[pl.ds(wid * BPW, BPW)])


BN = 2048
NGRID = VOCAB // BN
_DIMS = (((1,), (1,)), ((), ()))


def _mm_main(h_ref, w_ref, b_ref, out_hbm, obuf, sem):
    i = pl.program_id(0)
    h = h_ref[...]
    w = w_ref[...]
    acc = lax.dot_general(h, w, _DIMS, precision=lax.Precision.DEFAULT,
                          preferred_element_type=jnp.float32)
    bias = b_ref[0, pl.ds(pl.multiple_of(i * BN, 128), BN)]
    val = acc + bias[None, :]
    for s in (0, 1):
        @pl.when(lax.rem(i, 2) == s)
        def _():
            @pl.when(i >= 2)
            def _():
                pltpu.make_async_copy(
                    obuf.at[s],
                    out_hbm.at[:, pl.ds(pl.multiple_of((i - 2) * BN, 128), BN)],
                    sem.at[s],
                ).wait()
            obuf[s] = val
            pltpu.make_async_copy(
                obuf.at[s],
                out_hbm.at[:, pl.ds(pl.multiple_of(i * BN, 128), BN)],
                sem.at[s],
            ).start()

    @pl.when(i == NGRID - 1)
    def _():
        for s in (0, 1):
            pltpu.make_async_copy(
                obuf.at[s], out_hbm.at[:, pl.ds(0, BN)], sem.at[s]
            ).wait()


def _mm_tail(h_ref, w_ref, b_ref, lin_ref, out_ref):
    del lin_ref
    h = h_ref[...]
    w = w_ref[...]
    acc = lax.dot_general(h, w, _DIMS, precision=lax.Precision.DEFAULT,
                          preferred_element_type=jnp.float32)
    out_ref[...] = acc + b_ref[...]


def _project(h, proj_weight, proj_bias):
    bias2d = proj_bias.reshape(1, VOCAB)
    main = pl.pallas_call(
        _mm_main,
        grid=(NGRID,),
        in_specs=[
            pl.BlockSpec((BATCH, DIM), lambda i: (0, 0)),
            pl.BlockSpec((BN, DIM), lambda i: (i, 0)),
            pl.BlockSpec((1, VOCAB), lambda i: (0, 0)),
        ],
        out_specs=pl.BlockSpec(memory_space=pl.ANY),
        out_shape=jax.ShapeDtypeStruct((BATCH, VOCAB), jnp.float32),
        scratch_shapes=[
            pltpu.VMEM((2, BATCH, BN), jnp.float32),
            pltpu.SemaphoreType.DMA((2,)),
        ],
    )(h, proj_weight, bias2d)
    return pl.pallas_call(
        _mm_tail,
        grid=(1,),
        in_specs=[
            pl.BlockSpec((BATCH, DIM), lambda i: (0, 0)),
            pl.BlockSpec((BN, DIM), lambda i: (NGRID, 0)),
            pl.BlockSpec((1, BN), lambda i: (0, NGRID)),
            pl.BlockSpec(memory_space=pl.ANY),
        ],
        out_specs=pl.BlockSpec((BATCH, BN), lambda i: (0, NGRID)),
        out_shape=jax.ShapeDtypeStruct((BATCH, VOCAB), jnp.float32),
        input_output_aliases={3: 0},
    )(h, proj_weight, bias2d, main)


@jax.jit
def kernel(indices, embed_weight, proj_weight, proj_bias):
    idx = indices.astype(jnp.int32)
    idx2 = jnp.stack([2 * idx, 2 * idx + 1], axis=-1).reshape(BATCH, SEQ2)
    idx2 = jnp.pad(idx2, ((0, 0), (0, SEQ2P - SEQ2))).reshape(BATCH * SEQ2P)
    table2 = embed_weight.reshape(2 * VOCAB, HALF)
    h = _pool(idx2, table2)
    return _project(h, proj_weight, proj_bias)


# speedup vs baseline: 1.0587x; 1.0587x over previous
import functools

import jax
import jax.numpy as jnp
from jax import lax
from jax.experimental import pallas as pl
from jax.experimental.pallas import tpu as pltpu
from jax.experimental.pallas import tpu_sc as plsc

VOCAB = 100000
DIM = 256
BATCH = 1024
SEQ = 50

NC = 2
NS = 16
NW = NC * NS
BPW = BATCH // NW
LANES = 16

HALF = 128
SEQ2 = 2 * SEQ

_MESH = plsc.VectorSubcoreMesh(core_axis_name="c", subcore_axis_name="s")


@functools.partial(
    pl.kernel,
    mesh=_MESH,
    out_type=jax.ShapeDtypeStruct((BATCH, DIM), jnp.float32),
    scratch_types=[
        pltpu.VMEM((BPW, SEQ2), jnp.int32),
        pltpu.VMEM((SEQ2, HALF), jnp.float32),
        pltpu.VMEM((1, DIM), jnp.float32),
        pltpu.SemaphoreType.DMA,
    ],
)
def _pool(idx_hbm, table_hbm, h_hbm, idx_v, rows_v, hrow_v, sem):
    wid = lax.axis_index("s") * NC + lax.axis_index("c")
    base = wid * BPW
    pltpu.sync_copy(idx_hbm.at[pl.ds(base, BPW)], idx_v)

    def row_body(r, carry):
        pltpu.async_copy(table_hbm.at[idx_v.at[r]], rows_v, sem).wait()
        for c in range(HALF // LANES):
            def lo(j, acc):
                return acc + rows_v[2 * j, pl.ds(c * LANES, LANES)]
            def hi(j, acc):
                return acc + rows_v[2 * j + 1, pl.ds(c * LANES, LANES)]
            acc_lo = lax.fori_loop(0, SEQ, lo, jnp.zeros((LANES,), jnp.float32))
            acc_hi = lax.fori_loop(0, SEQ, hi, jnp.zeros((LANES,), jnp.float32))
            hrow_v[0, pl.ds(c * LANES, LANES)] = acc_lo * (1.0 / SEQ)
            hrow_v[0, pl.ds(HALF + c * LANES, LANES)] = acc_hi * (1.0 / SEQ)
        pltpu.sync_copy(hrow_v, h_hbm.at[pl.ds(base + r, 1)])
        return carry

    lax.fori_loop(0, BPW, row_body, 0)


def _mm_body(h_ref, w_ref, b_ref, out_ref):
    out_ref[...] = (
        lax.dot_general(
            h_ref[...], w_ref[...],
            (((1,), (1,)), ((), ())),
            preferred_element_type=jnp.float32,
        )
        + b_ref[...]
    )


def _project(h, proj_weight, proj_bias, bn=2048):
    nblk = (VOCAB + bn - 1) // bn
    return pl.pallas_call(
        _mm_body,
        grid=(nblk,),
        in_specs=[
            pl.BlockSpec((BATCH, DIM), lambda i: (0, 0)),
            pl.BlockSpec((bn, DIM), lambda i: (i, 0)),
            pl.BlockSpec((1, bn), lambda i: (0, i)),
        ],
        out_specs=pl.BlockSpec((BATCH, bn), lambda i: (0, i)),
        out_shape=jax.ShapeDtypeStruct((BATCH, VOCAB), jnp.float32),
    )(h, proj_weight, proj_bias.reshape(1, VOCAB))


@jax.jit
def kernel(indices, embed_weight, proj_weight, proj_bias):
    idx = indices.astype(jnp.int32)
    idx2 = jnp.stack([2 * idx, 2 * idx + 1], axis=-1).reshape(BATCH, SEQ2)
    table2 = embed_weight.reshape(2 * VOCAB, HALF)
    h = _pool(idx2, table2)
    return _project(h, proj_weight, proj_bias)
